# baseline (speedup 1.0000x reference)
import functools

import jax
import jax.numpy as jnp
from jax import lax
from jax.experimental import pallas as pl
from jax.experimental.pallas import tpu as pltpu
from jax.experimental.pallas import tpu_sc as plsc

_DIM = 64
_CHUNK = 128
_K = 8
_NC = 2
_NS = 16
_NW = _NC * _NS


def _embed_body(table_hbm, idx_hbm, out_hbm, idx_v, rows_v, sem):
    wid = lax.axis_index("s") * _NC + lax.axis_index("c")
    rows_total = idx_hbm.shape[0]
    per_w = rows_total // _NW
    base = wid * per_w
    n_groups = per_w // _K

    def body(g, carry):
        rb = base + g * _K
        pltpu.sync_copy(idx_hbm.at[pl.ds(rb, _K)], idx_v)
        handles = [
            pltpu.async_copy(table_hbm.at[idx_v.at[j]], rows_v.at[j], sem)
            for j in range(_K)
        ]
        for h in handles:
            h.wait()
        pltpu.sync_copy(rows_v, out_hbm.at[pl.ds(rb, _K)])
        return carry

    lax.fori_loop(0, n_groups, body, 0)


@functools.partial(jax.jit, static_argnames=("n_rows",))
def _gather_rows(table, idx2d, n_rows):
    mesh = plsc.VectorSubcoreMesh(core_axis_name="c", subcore_axis_name="s")
    fn = functools.partial(
        pl.kernel,
        mesh=mesh,
        out_type=jax.ShapeDtypeStruct((n_rows, _CHUNK, _DIM), jnp.float32),
        scratch_types=[
            pltpu.VMEM((_K, _CHUNK), jnp.int32),
            pltpu.VMEM((_K, _CHUNK, _DIM), jnp.float32),
            pltpu.SemaphoreType.DMA,
        ],
        compiler_params=pltpu.CompilerParams(use_tc_tiling_on_sc=False),
    )(_embed_body)
    return fn(table, idx2d)


def kernel(input_ids, attention_mask, table):
    b, s = input_ids.shape
    total = b * s
    n_rows = total // _CHUNK
    idx2d = input_ids.reshape(n_rows, _CHUNK)
    out3d = _gather_rows(table, idx2d, n_rows)
    token_embeddings = out3d.reshape(b, s, _DIM)
    return (input_ids, token_embeddings, attention_mask)


# speedup vs baseline: 1.0200x; 1.0200x over previous
import functools

import jax
import jax.numpy as jnp
from jax import lax
from jax.experimental import pallas as pl
from jax.experimental.pallas import tpu as pltpu
from jax.experimental.pallas import tpu_sc as plsc

_DIM = 64
_CHUNK = 128
_K = 4
_NC = 2
_NS = 16
_NW = _NC * _NS


def _embed_body(table_hbm, idx_hbm, out_hbm, idx_v, rows_v, gsem, wsem):
    wid = lax.axis_index("s") * _NC + lax.axis_index("c")
    rows_total = idx_hbm.shape[0]
    per_w = rows_total // _NW
    base = wid * per_w
    n_groups = per_w // _K

    pltpu.sync_copy(idx_hbm.at[pl.ds(base, per_w)], idx_v)

    def fire_gathers(g, s):
        for j in range(_K):
            pltpu.async_copy(
                table_hbm.at[idx_v.at[g * _K + j]], rows_v.at[s].at[j], gsem
            )

    def wait_gathers(s):
        pltpu.make_async_copy(out_hbm.at[pl.ds(0, _K)], rows_v.at[s], gsem).wait()

    def fire_writeback(g, s):
        pltpu.async_copy(rows_v.at[s], out_hbm.at[pl.ds(base + g * _K, _K)], wsem)

    def wait_writeback(s):
        pltpu.make_async_copy(rows_v.at[s], out_hbm.at[pl.ds(0, _K)], wsem).wait()

    fire_gathers(0, 0)

    def pair_body(i, carry):
        for s in (0, 1):
            g = 2 * i + s
            nxt_exists = g + 1 < n_groups

            @pl.when(nxt_exists)
            def _fire_next():
                if s == 1:
                    wait_writeback(1 - s)
                else:

                    @pl.when(g >= 1)
                    def _():
                        wait_writeback(1 - s)

                fire_gathers(g + 1, 1 - s)

            wait_gathers(s)
            fire_writeback(g, s)
        return carry

    lax.fori_loop(0, n_groups // 2, pair_body, 0)
    wait_writeback(0)
    wait_writeback(1)


@functools.partial(jax.jit, static_argnames=("n_rows",))
def _gather_rows(table, idx2d, n_rows):
    mesh = plsc.VectorSubcoreMesh(core_axis_name="c", subcore_axis_name="s")
    fn = functools.partial(
        pl.kernel,
        mesh=mesh,
        out_type=jax.ShapeDtypeStruct((n_rows, _CHUNK, _DIM), jnp.float32),
        scratch_types=[
            pltpu.VMEM((n_rows // _NW, _CHUNK), jnp.int32),
            pltpu.VMEM((2, _K, _CHUNK, _DIM), jnp.float32),
            pltpu.SemaphoreType.DMA,
            pltpu.SemaphoreType.DMA,
        ],
        compiler_params=pltpu.CompilerParams(use_tc_tiling_on_sc=False),
    )(_embed_body)
    return fn(table, idx2d)


def kernel(input_ids, attention_mask, table):
    b, s = input_ids.shape
    total = b * s
    n_rows = total // _CHUNK
    idx2d = input_ids.reshape(n_rows, _CHUNK)
    out3d = _gather_rows(table, idx2d, n_rows)
    token_embeddings = out3d.reshape(b, s, _DIM)
    return (input_ids, token_embeddings, attention_mask)
